# Initial kernel scaffold; baseline (speedup 1.0000x reference)
import jax
import jax.numpy as jnp
from jax.experimental import pallas as pl


def kernel(edge_index, node_attr, batch, params):
    raise NotImplementedError("write your pallas kernel here")


# speedup vs baseline: 1.0013x; 1.0013x over previous
import jax
import jax.numpy as jnp
import numpy as np
from jax.experimental import pallas as pl
from jax.experimental.pallas import tpu as pltpu

N = 10000
E = 320000
C = 128
B = 50
NPG = 200
A = 512
H = 4
DH = C // H


def _bn_act_kernel(x_ref, g_ref, b_ref, o_ref, *, act):
    x = x_ref[...]
    mu = jnp.mean(x, axis=0, keepdims=True)
    var = jnp.mean((x - mu) ** 2, axis=0, keepdims=True)
    y = g_ref[...] * (x - mu) / jnp.sqrt(var + 1e-5) + b_ref[...]
    if act == "elu":
        y = jnp.where(y > 0, y, jnp.exp(jnp.minimum(y, 0.0)) - 1.0)
    elif act == "relu":
        y = jnp.maximum(y, 0.0)
    o_ref[...] = y


def _bn_act(x, p, act):
    return pl.pallas_call(
        lambda x_ref, g_ref, b_ref, o_ref: _bn_act_kernel(x_ref, g_ref, b_ref, o_ref, act=act),
        out_shape=jax.ShapeDtypeStruct(x.shape, x.dtype),
    )(x, p["g"].reshape(1, C), p["b"].reshape(1, C))


def _gat(x, src, dst, p):
    h = x @ p["W"].T
    a = (h * p["as"]).sum(-1)
    d = (h * p["ad"]).sum(-1)
    e = jax.nn.leaky_relu(a[src] + d[dst], 0.2)
    m = jax.ops.segment_max(e, dst, num_segments=N)
    ex = jnp.exp(e - m[dst])
    den = jax.ops.segment_sum(ex, dst, num_segments=N)
    alpha = ex / (den[dst] + 1e-16)
    out = jax.ops.segment_sum(alpha[:, None] * h[src], dst, num_segments=N)
    return out + p["b"]


def _lin(x, p):
    return x @ p["W"].T + p["b"]


def kernel(edge_index, node_attr, batch, params):
    ar = jnp.arange(N, dtype=edge_index.dtype)
    ei = jnp.concatenate([edge_index, jnp.stack([ar, ar])], axis=1)
    src, dst = ei[0], ei[1]
    x = _gat(node_attr, src, dst, params["gc0"])
    x = _bn_act(x, params["bn0"], "elu")
    x = _gat(x, src, dst, params["gc1"])
    x = _bn_act(x, params["bn1"], "relu")
    r = x
    x = _gat(x, src, dst, params["rb_gc1"])
    x = _bn_act(x, params["rb_bn1"], "elu")
    x = _gat(x, src, dst, params["rb_gc2"])
    x = _bn_act(x, params["rb_bn2"], "relu")
    x = jax.nn.relu(x + r)
    px = _bn_act(_lin(x, params["p_lin"]), params["p_bn"], "relu")
    pv = _lin(px, params["p_final"])[:, 0]
    pout = jnp.pad(pv.reshape(B, NPG), ((0, 0), (0, A - NPG)))
    p = jax.nn.log_softmax(pout, axis=1)
    vx = _bn_act(_lin(x, params["v_lin"]), params["v_bn"], "relu")
    vx = _lin(vx, params["v_final"])
    g = vx.reshape(B, NPG, C)
    mha = params["mha"]
    q = jnp.ones((C,), jnp.float32)
    qh = (q @ mha["Wq"].T + mha["bq"]).reshape(H, DH)
    kh = (g @ mha["Wk"].T + mha["bk"]).reshape(B, NPG, H, DH)
    vh = (g @ mha["Wv"].T + mha["bv"]).reshape(B, NPG, H, DH)
    sc = jnp.einsum("hd,bnhd->bhn", qh, kh) / np.sqrt(DH).astype(np.float32)
    at = jax.nn.softmax(sc, axis=-1)
    o = jnp.einsum("bhn,bnhd->bhd", at, vh).reshape(B, C)
    o = o @ mha["Wo"].T + mha["bo"]
    v = jnp.tanh(_lin(o, params["v_read"]))
    return p, v


# speedup vs baseline: 25.6034x; 25.5698x over previous
import functools

import jax
import jax.numpy as jnp
import numpy as np
from jax import lax
from jax.experimental import pallas as pl
from jax.experimental.pallas import tpu as pltpu, tpu_sc as plsc

N = 10000
E = 320000
C = 128
B = 50
NPG = 200
A = 512
H = 4
DH = C // H

NTP = 10240
EX = E + N
NW = 32
EPW = 10368
NCH = EPW // 128
EP = NW * EPW

_SC_MESH = plsc.VectorSubcoreMesh(core_axis_name="c", subcore_axis_name="s")


@functools.partial(
    pl.kernel,
    out_type=(
        jax.ShapeDtypeStruct((NW, NCH, 128), jnp.float32),
        jax.ShapeDtypeStruct((NW * NTP,), jnp.float32),
    ),
    mesh=_SC_MESH,
    scratch_types=[
        pltpu.VMEM((NCH, 128), jnp.int32),
        pltpu.VMEM((NCH, 128), jnp.int32),
        pltpu.VMEM((NTP,), jnp.float32),
        pltpu.VMEM((NTP,), jnp.float32),
        pltpu.VMEM((NTP,), jnp.float32),
        pltpu.VMEM((NCH, 128), jnp.float32),
    ],
    compiler_params=pltpu.CompilerParams(needs_layout_passes=False),
)
def _gat_scalar_sc(a_h, d_h, src_h, dst_h, ex_h, den_h,
                   src_v, dst_v, a_v, d_v, den_v, ex_v):
    cid = lax.axis_index("c")
    sid = lax.axis_index("s")
    wid = sid * 2 + cid

    pltpu.sync_copy(src_h.at[wid], src_v)
    pltpu.sync_copy(dst_h.at[wid], dst_v)
    pltpu.sync_copy(a_h.at[0], a_v)
    pltpu.sync_copy(d_h.at[0], d_v)

    def _zden(i, _):
        den_v[pl.ds(i * 16, 16)] = jnp.zeros((16,), jnp.float32)
        return 0
    lax.fori_loop(0, NTP // 16, _zden, 0)

    def _mx(i, carry):
        am, dm = carry
        am = jnp.maximum(am, a_v[pl.ds(i * 16, 16)])
        dm = jnp.maximum(dm, d_v[pl.ds(i * 16, 16)])
        return am, dm
    neg = jnp.full((16,), -3.0e38, jnp.float32)
    am, dm = lax.fori_loop(0, NTP // 16, _mx, (neg, neg))
    s = lax.reduce_max(am, (0,)) + lax.reduce_max(dm, (0,))
    m_shift = jnp.maximum(s, 0.2 * s)

    gbase = wid * EPW

    def _chunk(c, _):
        def _g16(g, _):
            s16 = src_v[c, pl.ds(g * 16, 16)]
            d16 = dst_v[c, pl.ds(g * 16, 16)]
            av = plsc.load_gather(a_v, [s16])
            dv = plsc.load_gather(d_v, [d16])
            sv = av + dv
            e = jnp.where(sv >= 0, sv, 0.2 * sv)
            ex = jnp.exp(e - m_shift)
            gidx = gbase + c * 128 + g * 16 + lax.iota(jnp.int32, 16)
            ex = jnp.where(gidx < EX, ex, 0.0)
            ex_v[c, pl.ds(g * 16, 16)] = ex
            plsc.addupdate_scatter(den_v, [d16], ex)
            return 0
        lax.fori_loop(0, 8, _g16, 0)
        return 0
    lax.fori_loop(0, NCH, _chunk, 0)

    pltpu.sync_copy(ex_v, ex_h.at[wid])
    pltpu.sync_copy(den_v, den_h.at[pl.ds(wid * NTP, NTP)])


_STRIPE = 640


@functools.partial(
    pl.kernel,
    out_type=jax.ShapeDtypeStruct((2, N, C), jnp.float32),
    mesh=_SC_MESH,
    scratch_types=[
        pltpu.VMEM((NCH, 128), jnp.int32),
        pltpu.VMEM((NCH, 128), jnp.int32),
        pltpu.VMEM((NCH, 128), jnp.float32),
        pltpu.VMEM((128, C), jnp.float32),
        pltpu.VMEM_SHARED((N, C), jnp.float32),
        pltpu.SemaphoreType.DMA,
    ],
    compiler_params=pltpu.CompilerParams(needs_layout_passes=False),
)
def _gat_rows_sc(h_h, src_h, dst_h, ex_h, num_h,
                 src_v, dst_v, ex_v, rows_v, numsp, sem):
    cid = lax.axis_index("c")
    sid = lax.axis_index("s")
    wid = sid * 2 + cid

    pltpu.sync_copy(src_h.at[wid], src_v)
    pltpu.sync_copy(dst_h.at[wid], dst_v)
    pltpu.sync_copy(ex_h.at[wid], ex_v)

    def _zrows(i, _):
        for j in range(C // 16):
            rows_v[i, pl.ds(j * 16, 16)] = jnp.zeros((16,), jnp.float32)
        return 0
    lax.fori_loop(0, 128, _zrows, 0)

    stripe = sid * _STRIPE
    for z in range(5):
        @pl.when(jnp.logical_or(sid < 15, z < 3))
        def _():
            pltpu.sync_copy(rows_v, numsp.at[pl.ds(stripe + z * 128, 128)])
    @pl.when(sid == 15)
    def _():
        pltpu.sync_copy(rows_v.at[pl.ds(0, 16)],
                        numsp.at[pl.ds(9984, 16)])
    plsc.subcore_barrier()

    def _chunk(c, _):
        pltpu.async_copy(h_h.at[src_v.at[c]], rows_v, sem).wait()

        def _scale(e, _):
            exb = plsc.load_gather(
                ex_v, [jnp.broadcast_to(c, (16,)).astype(jnp.int32),
                       jnp.broadcast_to(e, (16,)).astype(jnp.int32)])
            for j in range(C // 16):
                rows_v[e, pl.ds(j * 16, 16)] = (
                    rows_v[e, pl.ds(j * 16, 16)] * exb)
            return 0
        lax.fori_loop(0, 128, _scale, 0)

        pltpu.sync_copy(rows_v, numsp.at[dst_v.at[c]], add=True)
        return 0
    lax.fori_loop(0, NCH, _chunk, 0)

    plsc.subcore_barrier()

    @pl.when(sid < 15)
    def _():
        pltpu.sync_copy(numsp.at[pl.ds(stripe, _STRIPE)],
                        num_h.at[cid, pl.ds(stripe, _STRIPE)])

    @pl.when(sid == 15)
    def _():
        pltpu.sync_copy(numsp.at[pl.ds(15 * _STRIPE, 400)],
                        num_h.at[cid, pl.ds(15 * _STRIPE, 400)])


def _pre_body(x_ref, w_ref, as_ref, ad_ref, h_ref, a_ref, d_ref):
    h = jnp.dot(x_ref[...], w_ref[...].T,
                preferred_element_type=jnp.float32)
    h_ref[...] = h
    a_ref[...] = jnp.sum(h * as_ref[...], axis=1).reshape(1, NTP)
    d_ref[...] = jnp.sum(h * ad_ref[...], axis=1).reshape(1, NTP)


def _gat_pre(x, p):
    return pl.pallas_call(
        _pre_body,
        out_shape=(
            jax.ShapeDtypeStruct((NTP, C), jnp.float32),
            jax.ShapeDtypeStruct((1, NTP), jnp.float32),
            jax.ShapeDtypeStruct((1, NTP), jnp.float32),
        ),
    )(x, p["W"], p["as"].reshape(1, C), p["ad"].reshape(1, C))


def _post_body(num_ref, den_ref, b_ref, g_ref, bb_ref, r_ref, o_ref, *, act):
    num = num_ref[0] + num_ref[1]
    den = den_ref[...].reshape(NW, NTP).sum(axis=0)[:N].reshape(N, 1)
    x = num / (den + 1e-16) + b_ref[...]
    mu = jnp.mean(x, axis=0, keepdims=True)
    var = jnp.mean((x - mu) ** 2, axis=0, keepdims=True)
    y = g_ref[...] * (x - mu) / jnp.sqrt(var + 1e-5) + bb_ref[...]
    if act == "elu":
        y = jnp.where(y > 0, y, jnp.exp(jnp.minimum(y, 0.0)) - 1.0)
    else:
        y = jnp.maximum(y, 0.0)
    if r_ref is not None:
        y = jnp.maximum(y + r_ref[...][:N], 0.0)
    o_ref[...] = jnp.concatenate(
        [y, jnp.zeros((NTP - N, C), jnp.float32)], axis=0)


def _gat_post(num, den, p, bn, act, resid=None):
    args = [num, den, p["b"].reshape(1, C), bn["g"].reshape(1, C),
            bn["b"].reshape(1, C)]
    if resid is None:
        body = lambda n_, d_, b_, g_, bb_, o_: _post_body(
            n_, d_, b_, g_, bb_, None, o_, act=act)
    else:
        args.append(resid)
        body = lambda n_, d_, b_, g_, bb_, r_, o_: _post_body(
            n_, d_, b_, g_, bb_, r_, o_, act=act)
    return pl.pallas_call(
        body,
        out_shape=jax.ShapeDtypeStruct((NTP, C), jnp.float32),
    )(*args)


def _heads1_body(x_ref, pw_ref, pb_ref, pg_ref, pbb_ref, pfw_ref, pfb_ref,
                 vw_ref, vb_ref, vg_ref, vbb_ref, vfw_ref, vfb_ref,
                 pv_ref, vxf_ref):
    x = x_ref[...][:N]

    def bnrelu(y, g, bb):
        mu = jnp.mean(y, axis=0, keepdims=True)
        var = jnp.mean((y - mu) ** 2, axis=0, keepdims=True)
        return jnp.maximum(g * (y - mu) / jnp.sqrt(var + 1e-5) + bb, 0.0)

    px = bnrelu(jnp.dot(x, pw_ref[...].T, preferred_element_type=jnp.float32)
                + pb_ref[...], pg_ref[...], pbb_ref[...])
    pv = jnp.sum(px * pfw_ref[...], axis=1) + pfb_ref[0, 0]
    pv_ref[...] = jnp.concatenate(
        [pv, jnp.zeros((NTP - N,), jnp.float32)]).reshape(1, NTP)

    vx = bnrelu(jnp.dot(x, vw_ref[...].T, preferred_element_type=jnp.float32)
                + vb_ref[...], vg_ref[...], vbb_ref[...])
    vxf = jnp.dot(vx, vfw_ref[...].T, preferred_element_type=jnp.float32) \
        + vfb_ref[...]
    vxf_ref[...] = jnp.concatenate(
        [vxf, jnp.zeros((NTP - N, C), jnp.float32)], axis=0)


def _heads1(x, params):
    return pl.pallas_call(
        _heads1_body,
        out_shape=(
            jax.ShapeDtypeStruct((1, NTP), jnp.float32),
            jax.ShapeDtypeStruct((NTP, C), jnp.float32),
        ),
    )(x,
      params["p_lin"]["W"], params["p_lin"]["b"].reshape(1, C),
      params["p_bn"]["g"].reshape(1, C), params["p_bn"]["b"].reshape(1, C),
      params["p_final"]["W"].reshape(1, C),
      params["p_final"]["b"].reshape(1, 1),
      params["v_lin"]["W"], params["v_lin"]["b"].reshape(1, C),
      params["v_bn"]["g"].reshape(1, C), params["v_bn"]["b"].reshape(1, C),
      params["v_final"]["W"], params["v_final"]["b"].reshape(1, C))


def _heads2_body(pout_ref, g_ref, wq_ref, bq_ref, wk_ref, bk_ref,
                 wv_ref, bv_ref, wo_ref, bo_ref, vrw_ref, vrb_ref,
                 p_ref, v_ref):
    pout = pout_ref[...]
    rm = jnp.max(pout, axis=1, keepdims=True)
    z = pout - rm
    p_ref[...] = z - jnp.log(jnp.sum(jnp.exp(z), axis=1, keepdims=True))

    g = g_ref[...]
    qh = jnp.sum(wq_ref[...], axis=1).reshape(1, C) + bq_ref[...]
    kh = jnp.dot(g, wk_ref[...].T, preferred_element_type=jnp.float32) \
        + bk_ref[...]
    vh = jnp.dot(g, wv_ref[...].T, preferred_element_type=jnp.float32) \
        + bv_ref[...]

    jidx = lax.broadcasted_iota(jnp.int32, (C, H), 0)
    hidx = lax.broadcasted_iota(jnp.int32, (C, H), 1)
    msum = (jidx // DH == hidx).astype(jnp.float32)
    bidx = lax.broadcasted_iota(jnp.int32, (B, N), 0)
    iidx = lax.broadcasted_iota(jnp.int32, (B, N), 1)
    gsel = (iidx // NPG == bidx).astype(jnp.float32)

    scn = jnp.dot(kh * qh, msum, preferred_element_type=jnp.float32) \
        / np.sqrt(DH).astype(np.float32)
    gmax = jnp.max(scn)
    eu = jnp.exp(scn - gmax)
    denom = jnp.dot(gsel, eu, preferred_element_type=jnp.float32)
    denb = jnp.dot(gsel.T, denom, preferred_element_type=jnp.float32)
    at = eu / denb
    atex = jnp.dot(at, msum.T, preferred_element_type=jnp.float32)
    o = jnp.dot(gsel, atex * vh, preferred_element_type=jnp.float32)
    o = jnp.dot(o, wo_ref[...].T, preferred_element_type=jnp.float32) \
        + bo_ref[...]
    v_ref[...] = jnp.tanh(
        jnp.sum(o * vrw_ref[...], axis=1, keepdims=True) + vrb_ref[0, 0])


def _heads2(pout, gfeat, mha, v_read):
    return pl.pallas_call(
        _heads2_body,
        out_shape=(
            jax.ShapeDtypeStruct((B, A), jnp.float32),
            jax.ShapeDtypeStruct((B, 1), jnp.float32),
        ),
    )(pout, gfeat,
      mha["Wq"], mha["bq"].reshape(1, C),
      mha["Wk"], mha["bk"].reshape(1, C),
      mha["Wv"], mha["bv"].reshape(1, C),
      mha["Wo"], mha["bo"].reshape(1, C),
      v_read["W"].reshape(1, C), v_read["b"].reshape(1, 1))


def _gat_layer(x, edge_args, p):
    h, a, d = _gat_pre(x, p)
    src_w, dst_w = edge_args
    ex, den = _gat_scalar_sc(a, d, src_w, dst_w)
    num = _gat_rows_sc(h, src_w, dst_w, ex)
    return num, den


def kernel(edge_index, node_attr, batch, params):
    ar = jnp.arange(N, dtype=jnp.int32)
    pad = jnp.zeros((EP - EX,), jnp.int32)
    srcx = jnp.concatenate([edge_index[0].astype(jnp.int32), ar, pad])
    dstx = jnp.concatenate([edge_index[1].astype(jnp.int32), ar, pad])
    src_w = srcx.reshape(NW, NCH, 128)
    dst_w = dstx.reshape(NW, NCH, 128)
    edge_args = (src_w, dst_w)

    x = jnp.pad(node_attr, ((0, NTP - N), (0, 0)))

    num, den = _gat_layer(x, edge_args, params["gc0"])
    x = _gat_post(num, den, params["gc0"], params["bn0"], "elu")
    num, den = _gat_layer(x, edge_args, params["gc1"])
    x = _gat_post(num, den, params["gc1"], params["bn1"], "relu")
    r = x
    num, den = _gat_layer(x, edge_args, params["rb_gc1"])
    x = _gat_post(num, den, params["rb_gc1"], params["rb_bn1"], "elu")
    num, den = _gat_layer(x, edge_args, params["rb_gc2"])
    x = _gat_post(num, den, params["rb_gc2"], params["rb_bn2"], "relu",
                  resid=r)

    pv, vxf = _heads1(x, params)
    pout = jnp.pad(pv[0, :N].reshape(B, NPG), ((0, 0), (0, A - NPG)))
    p, v = _heads2(pout, vxf[:N], params["mha"], params["v_read"])
    return p, v
